# baseline (speedup 1.0000x reference)
import functools

import jax
import jax.numpy as jnp
from jax.experimental import pallas as pl
from jax.experimental.pallas import tpu as pltpu

N = 4096
NH = 128
NL = 6
BR = 256
NB = N // BR
SPG = NL * NB
T_MM = 2 * SPG


def _mlp_block(t, L1w_ref, L1b_ref, L2w_ref, L2b_ref, L3wT_ref, L3b_ref):
    h1 = jnp.dot(t.astype(jnp.bfloat16), L1w_ref[...],
                 preferred_element_type=jnp.float32) + L1b_ref[...]
    h1 = jnp.maximum(h1, 0.0)
    h2 = jnp.dot(h1.astype(jnp.bfloat16), L2w_ref[...],
                 preferred_element_type=jnp.float32) + L2b_ref[...]
    h2 = jnp.maximum(h2, 0.0)
    return (jnp.sum(h2 * L3wT_ref[...], axis=1, keepdims=True)
            + L3b_ref[0, 0])


def _gnn_kernel(adj1_ref, adj2_ref, W1_ref, Wstack_ref,
                L1w_ref, L1b_ref, L2w_ref, L2b_ref, L3wT_ref, L3b_ref,
                out_ref,
                adj_sc, y_sc, c_sc, r_sc, s_sc, s1_sc, z_sc):
    t = pl.program_id(0)
    mlp = functools.partial(_mlp_block, L1w_ref=L1w_ref, L1b_ref=L1b_ref,
                            L2w_ref=L2w_ref, L2b_ref=L2b_ref,
                            L3wT_ref=L3wT_ref, L3b_ref=L3b_ref)

    @pl.when(t >= 1)
    def _():
        te = t - 1
        ge = te // SPG
        le = (te % SPG) // NB
        ie = te % NB
        rows = pl.ds(ie * BR, BR)
        z = z_sc[te % 2]
        act = jnp.maximum(z, 0.0)
        nrm2 = jnp.sum(act * act, axis=1, keepdims=True)
        scale = jax.lax.rsqrt(jnp.maximum(nrm2, 1e-24))
        n = jnp.where(le < NL - 1, act * scale, act)
        c_sc[rows, :] = n
        r_new = jnp.where(le == 0, 0.0, r_sc[rows, :]) + n
        r_sc[rows, :] = r_new
        s_new = jnp.where(le == 0, 0.0, s_sc[rows, :]) + mlp(n)
        s_sc[rows, :] = s_new

        @pl.when(le == NL - 1)
        def _():
            s_fin = s_new + mlp(r_new)

            @pl.when(ge == 0)
            def _():
                s1_sc[rows, :] = s_fin

            @pl.when(ge == 1)
            def _():
                out_ref[rows, :] = s_fin * s1_sc[rows, :] * (1.0 / 49.0)

    @pl.when(t < T_MM)
    def _():
        l = (t % SPG) // NB
        i = t % NB
        rows = pl.ds(i * BR, BR)

        @pl.when(t < NB)
        def _():
            adj_sc[rows, :] = adj1_ref[...].astype(jnp.bfloat16)

        @pl.when((i == 0) & (l == 0))
        def _():
            y_sc[...] = W1_ref[...]

        @pl.when((i == 0) & (l >= 1))
        def _():
            W = Wstack_ref[l - 1]
            y_sc[...] = jnp.dot(c_sc[...].astype(jnp.bfloat16), W,
                                preferred_element_type=jnp.float32
                                ).astype(jnp.bfloat16)

        z_sc[t % 2] = jnp.dot(adj_sc[rows, :], y_sc[...],
                              preferred_element_type=jnp.float32)

        @pl.when((t >= SPG - NB) & (t < SPG))
        def _():
            adj_sc[rows, :] = adj2_ref[...].astype(jnp.bfloat16)


def kernel(adj1, adj2, W1, W2, W3, W4, W5, W6, L1w, L1b, L2w, L2b, L3w, L3b):
    bf = jnp.bfloat16
    W1c = W1.astype(bf)
    Wstack = jnp.stack([W2, W3, W4, W5, W6]).astype(bf)
    L1wc = L1w.astype(bf)
    L2wc = L2w.astype(bf)
    L1b2 = L1b.reshape(1, -1)
    L2b2 = L2b.reshape(1, -1)
    L3wT = L3w.reshape(1, -1)
    L3b2 = L3b.reshape(1, 1)
    grid = (T_MM + 1,)
    adj2_start = SPG - NB
    return pl.pallas_call(
        _gnn_kernel,
        grid=grid,
        in_specs=[
            pl.BlockSpec((BR, N), lambda t: (jnp.minimum(t, NB - 1), 0)),
            pl.BlockSpec((BR, N),
                         lambda t: (jnp.clip(t - adj2_start, 0, NB - 1), 0)),
            pl.BlockSpec((N, NH), lambda t: (0, 0)),
            pl.BlockSpec((NL - 1, NH, NH), lambda t: (0, 0, 0)),
            pl.BlockSpec((NH, 2 * NH), lambda t: (0, 0)),
            pl.BlockSpec((1, 2 * NH), lambda t: (0, 0)),
            pl.BlockSpec((2 * NH, 2 * NH), lambda t: (0, 0)),
            pl.BlockSpec((1, 2 * NH), lambda t: (0, 0)),
            pl.BlockSpec((1, 2 * NH), lambda t: (0, 0)),
            pl.BlockSpec((1, 1), lambda t: (0, 0)),
        ],
        out_specs=pl.BlockSpec((N, 1), lambda t: (0, 0)),
        out_shape=jax.ShapeDtypeStruct((N, 1), jnp.float32),
        scratch_shapes=[
            pltpu.VMEM((N, N), jnp.bfloat16),
            pltpu.VMEM((N, NH), jnp.bfloat16),
            pltpu.VMEM((N, NH), jnp.float32),
            pltpu.VMEM((N, NH), jnp.float32),
            pltpu.VMEM((N, 1), jnp.float32),
            pltpu.VMEM((N, 1), jnp.float32),
            pltpu.VMEM((2, BR, NH), jnp.float32),
        ],
        compiler_params=pltpu.CompilerParams(
            dimension_semantics=("arbitrary",),
            vmem_limit_bytes=63 * 1024 * 1024,
        ),
    )(adj1, adj2, W1c, Wstack, L1wc, L1b2, L2wc, L2b2, L3wT, L3b2)


# speedup vs baseline: 1.2690x; 1.2690x over previous
import functools

import jax
import jax.numpy as jnp
from jax.experimental import pallas as pl
from jax.experimental.pallas import tpu as pltpu

N = 4096
NH = 128
NL = 6
BR = 512
NB = N // BR
T_MM = NL * NB


def _mlp_block(t, L1w_ref, L1b_ref, L2w_ref, L2b_ref, L3wT_ref, L3b_ref):
    h1 = jnp.dot(t.astype(jnp.bfloat16), L1w_ref[...],
                 preferred_element_type=jnp.float32) + L1b_ref[...]
    h1 = jnp.maximum(h1, 0.0)
    h2 = jnp.dot(h1.astype(jnp.bfloat16), L2w_ref[...],
                 preferred_element_type=jnp.float32) + L2b_ref[...]
    h2 = jnp.maximum(h2, 0.0)
    return (jnp.sum(h2 * L3wT_ref[...], axis=1, keepdims=True)
            + L3b_ref[0, 0])


def _gnn_kernel(adj_ref, s_other_ref, W1_ref, Wstack_ref,
                L1w_ref, L1b_ref, L2w_ref, L2b_ref, L3wT_ref, L3b_ref,
                out_ref,
                adj_sc, y_sc, c_sc, r_sc, s_sc, z_sc):
    t = pl.program_id(0)
    mlp = functools.partial(_mlp_block, L1w_ref=L1w_ref, L1b_ref=L1b_ref,
                            L2w_ref=L2w_ref, L2b_ref=L2b_ref,
                            L3wT_ref=L3wT_ref, L3b_ref=L3b_ref)

    @pl.when(t >= 1)
    def _():
        te = t - 1
        le = te // NB
        ie = te % NB
        rows = pl.ds(ie * BR, BR)
        z = z_sc[te % 2]
        act = jnp.maximum(z, 0.0)
        nrm2 = jnp.sum(act * act, axis=1, keepdims=True)
        scale = jax.lax.rsqrt(jnp.maximum(nrm2, 1e-24))
        n = jnp.where(le < NL - 1, act * scale, act)
        c_sc[rows, :] = n
        r_new = jnp.where(le == 0, 0.0, r_sc[rows, :]) + n
        r_sc[rows, :] = r_new
        s_new = jnp.where(le == 0, 0.0, s_sc[rows, :]) + mlp(n)
        s_sc[rows, :] = s_new

        @pl.when(le == NL - 1)
        def _():
            s_fin = s_new + mlp(r_new)
            out_ref[rows, :] = s_fin * s_other_ref[rows, :] * (1.0 / 7.0)

    @pl.when(t < T_MM)
    def _():
        l = t // NB
        i = t % NB
        rows = pl.ds(i * BR, BR)

        @pl.when(t < NB)
        def _():
            adj_sc[rows, :] = adj_ref[...].astype(jnp.bfloat16)

        @pl.when((i == 0) & (l == 0))
        def _():
            y_sc[...] = W1_ref[...]

        @pl.when((i == 0) & (l >= 1))
        def _():
            W = Wstack_ref[l - 1]
            y_sc[...] = jnp.dot(c_sc[...].astype(jnp.bfloat16), W,
                                preferred_element_type=jnp.float32
                                ).astype(jnp.bfloat16)

        z_sc[t % 2] = jnp.dot(adj_sc[rows, :], y_sc[...],
                              preferred_element_type=jnp.float32)


def _gnn_graph(adj, s_other, W1c, Wstack, L1wc, L1b2, L2wc, L2b2, L3wT, L3b2):
    return pl.pallas_call(
        _gnn_kernel,
        grid=(T_MM + 1,),
        in_specs=[
            pl.BlockSpec((BR, N), lambda t: (jnp.minimum(t, NB - 1), 0)),
            pl.BlockSpec((N, 1), lambda t: (0, 0)),
            pl.BlockSpec((N, NH), lambda t: (0, 0)),
            pl.BlockSpec((NL - 1, NH, NH), lambda t: (0, 0, 0)),
            pl.BlockSpec((NH, 2 * NH), lambda t: (0, 0)),
            pl.BlockSpec((1, 2 * NH), lambda t: (0, 0)),
            pl.BlockSpec((2 * NH, 2 * NH), lambda t: (0, 0)),
            pl.BlockSpec((1, 2 * NH), lambda t: (0, 0)),
            pl.BlockSpec((1, 2 * NH), lambda t: (0, 0)),
            pl.BlockSpec((1, 1), lambda t: (0, 0)),
        ],
        out_specs=pl.BlockSpec((N, 1), lambda t: (0, 0)),
        out_shape=jax.ShapeDtypeStruct((N, 1), jnp.float32),
        scratch_shapes=[
            pltpu.VMEM((N, N), jnp.bfloat16),
            pltpu.VMEM((N, NH), jnp.bfloat16),
            pltpu.VMEM((N, NH), jnp.float32),
            pltpu.VMEM((N, NH), jnp.float32),
            pltpu.VMEM((N, 1), jnp.float32),
            pltpu.VMEM((2, BR, NH), jnp.float32),
        ],
        compiler_params=pltpu.CompilerParams(
            dimension_semantics=("arbitrary",),
            vmem_limit_bytes=63 * 1024 * 1024,
        ),
    )(adj, s_other, W1c, Wstack, L1wc, L1b2, L2wc, L2b2, L3wT, L3b2)


def kernel(adj1, adj2, W1, W2, W3, W4, W5, W6, L1w, L1b, L2w, L2b, L3w, L3b):
    bf = jnp.bfloat16
    W1c = W1.astype(bf)
    Wstack = jnp.stack([W2, W3, W4, W5, W6]).astype(bf)
    L1wc = L1w.astype(bf)
    L2wc = L2w.astype(bf)
    L1b2 = L1b.reshape(1, -1)
    L2b2 = L2b.reshape(1, -1)
    L3wT = L3w.reshape(1, -1)
    L3b2 = L3b.reshape(1, 1)
    ones = jnp.ones((N, 1), jnp.float32)
    s1 = _gnn_graph(adj1, ones, W1c, Wstack, L1wc, L1b2, L2wc, L2b2, L3wT, L3b2)
    return _gnn_graph(adj2, s1, W1c, Wstack, L1wc, L1b2, L2wc, L2b2, L3wT, L3b2)
